# baseline (speedup 1.0000x reference)
import functools

import numpy as np
import jax
import jax.numpy as jnp
from jax.experimental import pallas as pl
from jax.experimental.pallas import tpu as pltpu

_SUP = 1
_CLS = 50


def _norm(x, axis):
    n = jnp.linalg.norm(x, axis=axis, keepdims=True)
    return x / jnp.maximum(n, 1e-12)


def _gather(t, idx):
    b = t.shape[0]
    return t[jnp.arange(b)[:, None, None], idx]


def _dist2(v):
    inner = jnp.einsum('bic,bjc->bij', v, v)
    quad = jnp.sum(v * v, axis=2)
    return -2.0 * inner + quad[:, None, :] + quad[:, :, None]


def _neighbor_table(verts, k):
    _, idx = jax.lax.top_k(-_dist2(verts), k)
    return idx


def _bn_relu(x):
    m = jnp.mean(x, axis=(0, 1), keepdims=True)
    v = jnp.var(x, axis=(0, 1), keepdims=True)
    return jax.nn.relu((x - m) / jnp.sqrt(v + 1e-5))


def _conv_surface(p, nbr_idx, verts):
    nd = _norm(_gather(verts, nbr_idx) - verts[:, :, None, :], -1)
    sd = _norm(p['d'], 0)
    theta = jax.nn.relu(jnp.einsum('bvnc,ck->bvnk', nd, sd))
    return jnp.max(theta, axis=2)


def _conv_layer(p, nbr_idx, verts, fmap, out_c):
    nd = _norm(_gather(verts, nbr_idx) - verts[:, :, None, :], -1)
    sd = _norm(p['d'], 0)
    theta = jax.nn.relu(jnp.einsum('bvnc,ck->bvnk', nd, sd))
    fout = fmap @ p['w'] + p['b']
    fc = fout[:, :, :out_c]
    fs = _gather(fout[:, :, out_c:], nbr_idx)
    return fc + jnp.max(theta * fs, axis=2)


def _fusion(p, nl, ng, verts, feat, dim):
    fm_l = _bn_relu(_conv_layer(p['l'], nl, verts, feat, dim))
    fm_g = _bn_relu(_conv_layer(p['g0'], ng, verts, feat, dim))
    fm_g = _bn_relu(_conv_layer(p['g1'], ng, verts, fm_g, dim))
    return jnp.concatenate([fm_l, fm_g], axis=2)


def _tblock(p, xyz, x, idx16):
    knn_xyz = _gather(xyz, idx16)
    knn_f = _gather(x, idx16)
    pos = xyz[:, :, None, :] - knn_xyz
    pos_enc = jax.nn.relu(pos @ p['w1'] + p['b1']) @ p['w2'] + p['b2']
    xq = knn_f @ p['wqk'] + p['bqk']
    energy = pos_enc + xq * xq
    att = jax.nn.softmax(energy, axis=-1)
    att = att / (1e-09 + jnp.sum(att, axis=1, keepdims=True))
    xv = knn_f @ p['wv'] + p['bv']
    return x + jnp.sum(att * xv, axis=2)


def _nearest(target, source):
    inner = jnp.einsum('bic,bjc->bij', target, source)
    d = (jnp.sum(source * source, axis=2)[:, None, :]
         + jnp.sum(target * target, axis=2)[:, :, None] - 2.0 * inner)
    return jnp.argmin(d, axis=-1)


def _head_body(x_ref, w1_ref, b1_ref, w2_ref, b2_ref, w3_ref, b3_ref, o_ref):
    x = x_ref[...]
    h = jnp.maximum(jnp.dot(x, w1_ref[...],
                            preferred_element_type=jnp.float32) + b1_ref[...], 0.0)
    h = jnp.maximum(jnp.dot(h, w2_ref[...],
                            preferred_element_type=jnp.float32) + b2_ref[...], 0.0)
    o = jnp.dot(h, w3_ref[...], preferred_element_type=jnp.float32) + b3_ref[...]
    m = jnp.max(o, axis=-1, keepdims=True)
    s = o - m
    lse = jnp.log(jnp.sum(jnp.exp(s), axis=-1, keepdims=True))
    o_ref[...] = s - lse


def _head(fuse, c1, c2, c3):
    n, d = fuse.shape
    blk = 256
    grid = (n // blk,)
    out = pl.pallas_call(
        _head_body,
        grid=grid,
        in_specs=[
            pl.BlockSpec((blk, d), lambda i: (i, 0)),
            pl.BlockSpec((d, 512), lambda i: (0, 0)),
            pl.BlockSpec((1, 512), lambda i: (0, 0)),
            pl.BlockSpec((512, 512), lambda i: (0, 0)),
            pl.BlockSpec((1, 512), lambda i: (0, 0)),
            pl.BlockSpec((512, _CLS), lambda i: (0, 0)),
            pl.BlockSpec((1, _CLS), lambda i: (0, 0)),
        ],
        out_specs=pl.BlockSpec((blk, _CLS), lambda i: (i, 0)),
        out_shape=jax.ShapeDtypeStruct((n, _CLS), jnp.float32),
    )(fuse, c1['w'], c1['b'][None, :], c2['w'], c2['b'][None, :],
      c3['w'], c3['b'][None, :])
    return out


def kernel(vertices, onehot, params):
    verts = jnp.transpose(vertices, (0, 2, 1))
    b, n, _ = verts.shape

    nbr0 = _neighbor_table(verts, 101)
    nl0, ng0 = nbr0[:, :, 1:11], nbr0[:, :, 1:101]
    tb0, p40 = nbr0[:, :, :16], nbr0[:, :, 1:5]

    c0 = params['conv_0']
    fm_l = _bn_relu(_conv_surface(c0['l'], nl0, verts))
    fm_g = _bn_relu(_conv_surface(c0['g0'], ng0, verts))
    fm_g = _bn_relu(_conv_layer(c0['g1'], ng0, verts, fm_g, 128))
    fm_0 = jnp.concatenate([fm_l, fm_g], axis=2)
    fm_0 = jax.nn.relu(fm_0 @ params['down0']['w'] + params['down0']['b'])
    fm_0 = _tblock(params['att0'], verts, fm_0, tb0)

    fm_1 = _fusion(params['conv_1'], nl0, ng0, verts, fm_0, 128)
    fm_1 = jax.nn.relu(fm_1 @ params['down1']['w'] + params['down1']['b'])
    fm_1 = _tblock(params['att1'], verts, fm_1, tb0)

    keep0 = np.random.RandomState(0).permutation(n)[: n // 4]
    pooled = jnp.max(_gather(fm_1, p40), axis=2)
    v1, fp1 = verts[:, keep0, :], pooled[:, keep0, :]

    nbr1 = _neighbor_table(v1, 101)
    nl1, ng1 = nbr1[:, :, 1:11], nbr1[:, :, 1:101]
    tb1, p41 = nbr1[:, :, :16], nbr1[:, :, 1:5]

    fm_2 = _fusion(params['conv_2'], nl1, ng1, v1, fp1, 128)
    fm_2 = _tblock(params['att2'], v1, fm_2, tb1)
    fm_3 = _fusion(params['conv_3'], nl1, ng1, v1, fm_2, 256)
    fm_3 = _tblock(params['att3'], v1, fm_3, tb1)

    keep1 = np.random.RandomState(1).permutation(v1.shape[1])[: v1.shape[1] // 4]
    pooled2 = jnp.max(_gather(fm_3, p41), axis=2)
    v2, fp2 = v1[:, keep1, :], pooled2[:, keep1, :]

    nbr2 = _neighbor_table(v2, 101)
    nl2, ng2 = nbr2[:, :, 1:11], nbr2[:, :, 1:101]
    tb2 = nbr2[:, :, :16]

    fm_4 = _fusion(params['conv_4'], nl2, ng2, v2, fp2, 512)
    fm_4 = jax.nn.relu(fm_4 @ params['down2']['w'] + params['down2']['b'])
    fm_4 = _tblock(params['att4'], v2, fm_4, tb2)

    f_global = jnp.max(fm_4, axis=1)
    ni1 = _nearest(verts, v1)
    ni2 = _nearest(verts, v2)
    bidx = jnp.arange(b)[:, None]
    fm_2u = fm_2[bidx, ni1]
    fm_3u = fm_3[bidx, ni1]
    fm_4u = fm_4[bidx, ni2]
    fg = jnp.broadcast_to(f_global[:, None, :], (b, n, f_global.shape[-1]))
    oh = jnp.broadcast_to(onehot[:, None, :], (b, n, onehot.shape[-1]))
    fuse = jnp.concatenate([fm_0, fm_1, fm_2u, fm_3u, fm_4u, fg, oh], axis=2)

    out = _head(fuse[0], params['c1'], params['c2'], params['c3'])
    return out[None]


# speedup vs baseline: 4.9801x; 4.9801x over previous
import functools

import numpy as np
import jax
import jax.numpy as jnp
from jax import lax
from jax.experimental import pallas as pl
from jax.experimental.pallas import tpu as pltpu
from jax.experimental.pallas import tpu_sc as plsc

_CLS = 50


def _pick_chunk(b_per_w, d):
    cap = max(8, min(128, (200 * 1024) // (d * 4) // 8 * 8))
    c = min(b_per_w, cap)
    while c > 8 and (b_per_w % c or c % 8):
        c -= 8
    if b_per_w % c:
        c = 8
    return c


@functools.lru_cache(maxsize=None)
def _sc_gather_fn(V, D, M):
    info = plsc.get_sparse_core_info()
    NC, NS = info.num_cores, info.num_subcores
    NW = NC * NS
    assert M % (8 * NW) == 0, (M, NW)
    b_per_w = M // NW
    chunk = _pick_chunk(b_per_w, D)
    nchunks = b_per_w // chunk
    mesh = plsc.VectorSubcoreMesh(core_axis_name="c", subcore_axis_name="s")

    @functools.partial(
        pl.kernel, mesh=mesh,
        compiler_params=pltpu.CompilerParams(use_tc_tiling_on_sc=False),
        out_type=jax.ShapeDtypeStruct((M, D), jnp.float32),
        scratch_types=[
            pltpu.VMEM((b_per_w,), jnp.int32),
            pltpu.VMEM((chunk, D), jnp.float32),
            pltpu.VMEM((chunk, D), jnp.float32),
            pltpu.SemaphoreType.DMA,
            pltpu.SemaphoreType.DMA,
        ],
    )
    def k(table_hbm, idx_hbm, out_hbm, idx_v, rows_a, rows_b, sem_a, sem_b):
        wid = lax.axis_index("s") * NC + lax.axis_index("c")
        base = wid * b_per_w
        pltpu.sync_copy(idx_hbm.at[pl.ds(base, b_per_w)], idx_v)
        bufs = [(rows_a, sem_a), (rows_b, sem_b)]
        cps = [None, None]

        def start(i):
            rows, sem = bufs[i % 2]
            cps[i % 2] = pltpu.async_copy(
                table_hbm.at[idx_v.at[pl.ds(i * chunk, chunk)]], rows, sem)

        start(0)
        for i in range(nchunks):
            if i + 1 < nchunks:
                start(i + 1)
            rows, _ = bufs[i % 2]
            cps[i % 2].wait()
            pltpu.sync_copy(rows, out_hbm.at[pl.ds(base + i * chunk, chunk)])

    return k


def _sc_gather(table, idx_flat):
    V, D = table.shape
    M = idx_flat.shape[0]
    return _sc_gather_fn(V, D, M)(table, idx_flat.astype(jnp.int32))


def _maxmul_body(th_ref, fs_ref, o_ref):
    o_ref[...] = jnp.max(th_ref[...] * fs_ref[...], axis=1)


def _max_body(th_ref, o_ref):
    o_ref[...] = jnp.max(th_ref[...], axis=1)


def _maxmul_reduce(theta, fs):
    V, Np, C = theta.shape
    P = 64
    return pl.pallas_call(
        _maxmul_body,
        grid=(V // P,),
        in_specs=[
            pl.BlockSpec((P, Np, C), lambda i: (i, 0, 0)),
            pl.BlockSpec((P, Np, C), lambda i: (i, 0, 0)),
        ],
        out_specs=pl.BlockSpec((P, C), lambda i: (i, 0)),
        out_shape=jax.ShapeDtypeStruct((V, C), jnp.float32),
    )(theta, fs)


def _max_reduce(theta):
    V, Np, C = theta.shape
    P = 64
    return pl.pallas_call(
        _max_body,
        grid=(V // P,),
        in_specs=[pl.BlockSpec((P, Np, C), lambda i: (i, 0, 0))],
        out_specs=pl.BlockSpec((P, C), lambda i: (i, 0)),
        out_shape=jax.ShapeDtypeStruct((V, C), jnp.float32),
    )(theta)


def _head_body(x_ref, w1_ref, b1_ref, w2_ref, b2_ref, w3_ref, b3_ref, o_ref):
    x = x_ref[...]
    h = jnp.maximum(jnp.dot(x, w1_ref[...],
                            preferred_element_type=jnp.float32) + b1_ref[...], 0.0)
    h = jnp.maximum(jnp.dot(h, w2_ref[...],
                            preferred_element_type=jnp.float32) + b2_ref[...], 0.0)
    o = jnp.dot(h, w3_ref[...], preferred_element_type=jnp.float32) + b3_ref[...]
    m = jnp.max(o, axis=-1, keepdims=True)
    s = o - m
    lse = jnp.log(jnp.sum(jnp.exp(s), axis=-1, keepdims=True))
    o_ref[...] = s - lse


def _head(fuse, c1, c2, c3):
    n, d = fuse.shape
    blk = 256
    return pl.pallas_call(
        _head_body,
        grid=(n // blk,),
        in_specs=[
            pl.BlockSpec((blk, d), lambda i: (i, 0)),
            pl.BlockSpec((d, 512), lambda i: (0, 0)),
            pl.BlockSpec((1, 512), lambda i: (0, 0)),
            pl.BlockSpec((512, 512), lambda i: (0, 0)),
            pl.BlockSpec((1, 512), lambda i: (0, 0)),
            pl.BlockSpec((512, _CLS), lambda i: (0, 0)),
            pl.BlockSpec((1, _CLS), lambda i: (0, 0)),
        ],
        out_specs=pl.BlockSpec((blk, _CLS), lambda i: (i, 0)),
        out_shape=jax.ShapeDtypeStruct((n, _CLS), jnp.float32),
    )(fuse, c1['w'], c1['b'][None, :], c2['w'], c2['b'][None, :],
      c3['w'], c3['b'][None, :])


def _norm(x, axis):
    n = jnp.linalg.norm(x, axis=axis, keepdims=True)
    return x / jnp.maximum(n, 1e-12)


def _bn_relu(x):
    m = jnp.mean(x, axis=(0, 1), keepdims=True)
    v = jnp.var(x, axis=(0, 1), keepdims=True)
    return jax.nn.relu((x - m) / jnp.sqrt(v + 1e-5))


def _pad_dup(idx, np_):
    n = idx.shape[1]
    if n == np_:
        return idx
    reps = [idx] + [idx[:, : min(n, np_ - n)]] * ((np_ - n + n - 1) // n)
    return jnp.concatenate(reps, axis=1)[:, :np_]


def _stage_idx(verts_b):
    inner = jnp.einsum('bic,bjc->bij', verts_b, verts_b)
    quad = jnp.sum(verts_b * verts_b, axis=2)
    dist = -2.0 * inner + quad[:, None, :] + quad[:, :, None]
    _, idx = lax.top_k(-dist, 101)
    idx = idx[0]
    nl = _pad_dup(idx[:, 1:11], 16)
    ng = _pad_dup(idx[:, 1:101], 104)
    tb = idx[:, :16]
    p4 = _pad_dup(idx[:, 1:5], 8)
    return nl, ng, tb, p4


def _gxyz(xyz16, idx_pad):
    V, Np = idx_pad.shape
    g = _sc_gather(xyz16, idx_pad.reshape(-1)).reshape(V, Np, 16)
    return g[None, :, :, :3]


def _nd(verts_b, gx):
    return _norm(gx - verts_b[:, :, None, :], -1)


def _theta(p, nd):
    sd = _norm(p['d'], 0)
    return jax.nn.relu(jnp.einsum('bvnc,ck->bvnk', nd, sd))


def _conv_layer(p, idx_pad, nd, fmap_b, out_c):
    theta = _theta(p, nd)
    fout = fmap_b @ p['w'] + p['b']
    fc = fout[:, :, :out_c]
    V, Np = idx_pad.shape
    fs = _sc_gather(fout[0, :, out_c:], idx_pad.reshape(-1)).reshape(V, Np, out_c)
    return fc + _maxmul_reduce(theta[0], fs)[None]


def _conv_surface(p, nd):
    return _max_reduce(_theta(p, nd)[0])[None]


def _fusion(p, nl, ng, ndl, ndg, feat_b, dim):
    fm_l = _bn_relu(_conv_layer(p['l'], nl, ndl, feat_b, dim))
    fm_g = _bn_relu(_conv_layer(p['g0'], ng, ndg, feat_b, dim))
    fm_g = _bn_relu(_conv_layer(p['g1'], ng, ndg, fm_g, dim))
    return jnp.concatenate([fm_l, fm_g], axis=2)


def _tblock(p, xyz16, verts_b, x_b, idx16):
    V = x_b.shape[1]
    C = x_b.shape[2]
    tab = jnp.concatenate([xyz16, x_b[0]], axis=1)
    g = _sc_gather(tab, idx16.reshape(-1)).reshape(V, 16, 16 + C)
    knn_xyz = g[None, :, :, :3]
    knn_f = g[None, :, :, 16:]
    pos = verts_b[:, :, None, :] - knn_xyz
    pos_enc = jax.nn.relu(pos @ p['w1'] + p['b1']) @ p['w2'] + p['b2']
    xq = knn_f @ p['wqk'] + p['bqk']
    energy = pos_enc + xq * xq
    att = jax.nn.softmax(energy, axis=-1)
    att = att / (1e-09 + jnp.sum(att, axis=1, keepdims=True))
    xv = knn_f @ p['wv'] + p['bv']
    return x_b + jnp.sum(att * xv, axis=2)


def _pool_gather(fmap_b, p4):
    V, Np = p4.shape
    C = fmap_b.shape[2]
    g = _sc_gather(fmap_b[0], p4.reshape(-1)).reshape(1, V, Np, C)
    return jnp.max(g, axis=2)


def _nearest(target_b, source_b):
    inner = jnp.einsum('bic,bjc->bij', target_b, source_b)
    d = (jnp.sum(source_b * source_b, axis=2)[:, None, :]
         + jnp.sum(target_b * target_b, axis=2)[:, :, None] - 2.0 * inner)
    return jnp.argmin(d, axis=-1)


def _forward(vertices, onehot, params):
    verts_b = jnp.transpose(vertices, (0, 2, 1))
    b, n, _ = verts_b.shape
    xyz0 = jnp.zeros((n, 16), jnp.float32).at[:, :3].set(verts_b[0])

    nl0, ng0, tb0, p40 = _stage_idx(verts_b)
    ndl0 = _nd(verts_b, _gxyz(xyz0, nl0))
    ndg0 = _nd(verts_b, _gxyz(xyz0, ng0))

    c0 = params['conv_0']
    fm_l = _bn_relu(_conv_surface(c0['l'], ndl0))
    fm_g = _bn_relu(_conv_surface(c0['g0'], ndg0))
    fm_g = _bn_relu(_conv_layer(c0['g1'], ng0, ndg0, fm_g, 128))
    fm_0 = jnp.concatenate([fm_l, fm_g], axis=2)
    fm_0 = jax.nn.relu(fm_0 @ params['down0']['w'] + params['down0']['b'])
    fm_0 = _tblock(params['att0'], xyz0, verts_b, fm_0, tb0)

    fm_1 = _fusion(params['conv_1'], nl0, ng0, ndl0, ndg0, fm_0, 128)
    fm_1 = jax.nn.relu(fm_1 @ params['down1']['w'] + params['down1']['b'])
    fm_1 = _tblock(params['att1'], xyz0, verts_b, fm_1, tb0)

    keep0 = np.random.RandomState(0).permutation(n)[: n // 4]
    pooled = _pool_gather(fm_1, p40)
    v1_b, fp1 = verts_b[:, keep0, :], pooled[:, keep0, :]
    xyz1 = xyz0[keep0]

    nl1, ng1, tb1, p41 = _stage_idx(v1_b)
    ndl1 = _nd(v1_b, _gxyz(xyz1, nl1))
    ndg1 = _nd(v1_b, _gxyz(xyz1, ng1))

    fm_2 = _fusion(params['conv_2'], nl1, ng1, ndl1, ndg1, fp1, 128)
    fm_2 = _tblock(params['att2'], xyz1, v1_b, fm_2, tb1)
    fm_3 = _fusion(params['conv_3'], nl1, ng1, ndl1, ndg1, fm_2, 256)
    fm_3 = _tblock(params['att3'], xyz1, v1_b, fm_3, tb1)

    V1 = n // 4
    keep1 = np.random.RandomState(1).permutation(V1)[: V1 // 4]
    pooled2 = _pool_gather(fm_3, p41)
    v2_b, fp2 = v1_b[:, keep1, :], pooled2[:, keep1, :]
    xyz2 = xyz1[keep1]

    nl2, ng2, tb2, _ = _stage_idx(v2_b)
    ndl2 = _nd(v2_b, _gxyz(xyz2, nl2))
    ndg2 = _nd(v2_b, _gxyz(xyz2, ng2))

    fm_4 = _fusion(params['conv_4'], nl2, ng2, ndl2, ndg2, fp2, 512)
    fm_4 = jax.nn.relu(fm_4 @ params['down2']['w'] + params['down2']['b'])
    fm_4 = _tblock(params['att4'], xyz2, v2_b, fm_4, tb2)

    f_global = jnp.max(fm_4, axis=1)
    ni1 = _nearest(verts_b, v1_b)
    ni2 = _nearest(verts_b, v2_b)
    bidx = jnp.arange(b)[:, None]
    fm_2u = fm_2[bidx, ni1]
    fm_3u = fm_3[bidx, ni1]
    fm_4u = fm_4[bidx, ni2]
    fg = jnp.broadcast_to(f_global[:, None, :], (b, n, f_global.shape[-1]))
    oh = jnp.broadcast_to(onehot[:, None, :], (b, n, onehot.shape[-1]))
    fuse = jnp.concatenate([fm_0, fm_1, fm_2u, fm_3u, fm_4u, fg, oh], axis=2)
    return _head(fuse[0], params['c1'], params['c2'], params['c3'])[None]


def kernel(vertices, onehot, params):
    return _forward(vertices, onehot, params)
